# baseline (speedup 1.0000x reference)
import jax
import jax.numpy as jnp
from jax.experimental import pallas as pl

_BLK = 512


def _bcast_body(w_ref, out_ref):
    out_ref[...] = jnp.broadcast_to(w_ref[...][None, :, :], out_ref.shape)


def kernel(tokens, W):
    b = tokens.shape[0]
    S, D = W.shape
    n_blk = S // _BLK
    return pl.pallas_call(
        _bcast_body,
        grid=(n_blk,),
        in_specs=[pl.BlockSpec((_BLK, D), lambda i: (i, 0))],
        out_specs=pl.BlockSpec((b, _BLK, D), lambda i: (0, i, 0)),
        out_shape=jax.ShapeDtypeStruct((b, S, D), W.dtype),
    )(W)
